# baseline (speedup 1.0000x reference)
import functools

import jax
import jax.numpy as jnp
from jax.experimental import pallas as pl
from jax.experimental.pallas import tpu as pltpu

N = 10000
C = 128
BM = 400


def _label_kernel(y_ref, mask_ref, label_ref):
    classes = jax.lax.broadcasted_iota(jnp.int32, (N, C), 1)
    eq = classes == y_ref[:]
    label_ref[:] = jnp.where(eq, mask_ref[:], 0.0)


def _matmul_kernel(label_ref, prop_ref, out_ref):
    acc = jax.lax.dot_general(
        prop_ref[:],
        label_ref[:],
        (((1,), (0,)), ((), ())),
        preferred_element_type=jnp.float32,
        precision=jax.lax.Precision.DEFAULT,
    )
    out_ref[:] = jnp.clip(acc, 0.0, 1.0)


@functools.partial(jax.jit, static_argnames=())
def kernel(x, y, train_mask, prop):
    del x
    y2 = y.reshape(N, 1)
    mask2 = train_mask.astype(jnp.float32).reshape(N, 1)
    label = pl.pallas_call(
        _label_kernel,
        out_shape=jax.ShapeDtypeStruct((N, C), jnp.float32),
    )(y2, mask2)
    return pl.pallas_call(
        _matmul_kernel,
        grid=(N // BM,),
        in_specs=[
            pl.BlockSpec((N, C), lambda i: (0, 0)),
            pl.BlockSpec((BM, N), lambda i: (i, 0)),
        ],
        out_specs=pl.BlockSpec((BM, C), lambda i: (i, 0)),
        out_shape=jax.ShapeDtypeStruct((N, C), jnp.float32),
        compiler_params=pltpu.CompilerParams(
            dimension_semantics=("parallel",),
        ),
    )(label, prop)


# speedup vs baseline: 1.0219x; 1.0219x over previous
import functools

import jax
import jax.numpy as jnp
from jax.experimental import pallas as pl
from jax.experimental.pallas import tpu as pltpu

N = 10000
C = 128
BM = 400
NS = 5
BS = BM // NS


def _lp_kernel(y_ref, mask_ref, p0, p1, p2, p3, p4, out_ref, label_ref):
    @pl.when(pl.program_id(0) == 0)
    def _build_label():
        classes = jax.lax.broadcasted_iota(jnp.int32, (N, C), 1)
        eq = classes == y_ref[:]
        label_ref[:] = jnp.where(eq, mask_ref[:], 0.0).astype(jnp.bfloat16)

    for k, p in enumerate((p0, p1, p2, p3, p4)):
        acc = jax.lax.dot_general(
            p[:].astype(jnp.bfloat16),
            label_ref[:],
            (((1,), (0,)), ((), ())),
            preferred_element_type=jnp.float32,
        )
        out_ref[k * BS:(k + 1) * BS, :] = jnp.clip(acc, 0.0, 1.0)


@functools.partial(jax.jit, static_argnames=())
def kernel(x, y, train_mask, prop):
    del x
    y2 = y.reshape(N, 1)
    mask2 = train_mask.astype(jnp.float32).reshape(N, 1)
    prop_specs = [
        pl.BlockSpec((BS, N), functools.partial(lambda k, i: (NS * i + k, 0), k))
        for k in range(NS)
    ]
    return pl.pallas_call(
        _lp_kernel,
        grid=(N // BM,),
        in_specs=[
            pl.BlockSpec((N, 1), lambda i: (0, 0)),
            pl.BlockSpec((N, 1), lambda i: (0, 0)),
            *prop_specs,
        ],
        out_specs=pl.BlockSpec((BM, C), lambda i: (i, 0)),
        out_shape=jax.ShapeDtypeStruct((N, C), jnp.float32),
        scratch_shapes=[pltpu.VMEM((N, C), jnp.bfloat16)],
        compiler_params=pltpu.CompilerParams(
            dimension_semantics=("arbitrary",),
        ),
    )(y2, mask2, prop, prop, prop, prop, prop)


# speedup vs baseline: 1.0294x; 1.0074x over previous
import functools

import jax
import jax.numpy as jnp
from jax.experimental import pallas as pl
from jax.experimental.pallas import tpu as pltpu

N = 10000
C = 128
BM = 200
NBUF = 4


def _lp_kernel(y_ref, mask_ref, prop_hbm, out_hbm, label_ref):
    classes = jax.lax.broadcasted_iota(jnp.int32, (N, C), 1)
    eq = classes == y_ref[:]
    label_ref[:] = jnp.where(eq, mask_ref[:], 0.0)

    def step(prop_blk, out_blk):
        acc = jax.lax.dot_general(
            prop_blk[:],
            label_ref[:],
            (((1,), (0,)), ((), ())),
            preferred_element_type=jnp.float32,
        )
        out_blk[:] = jnp.clip(acc, 0.0, 1.0)

    pipeline = pltpu.emit_pipeline(
        step,
        grid=(N // BM,),
        in_specs=[
            pl.BlockSpec((BM, N), lambda i: (i, 0),
                         pipeline_mode=pl.Buffered(buffer_count=NBUF)),
        ],
        out_specs=[pl.BlockSpec((BM, C), lambda i: (i, 0))],
    )
    pipeline(prop_hbm, out_hbm)


@functools.partial(jax.jit, static_argnames=())
def kernel(x, y, train_mask, prop):
    del x
    y2 = y.reshape(N, 1)
    mask2 = train_mask.astype(jnp.float32).reshape(N, 1)
    return pl.pallas_call(
        _lp_kernel,
        in_specs=[
            pl.BlockSpec((N, 1), lambda: (0, 0)),
            pl.BlockSpec((N, 1), lambda: (0, 0)),
            pl.BlockSpec(memory_space=pl.ANY),
        ],
        out_specs=pl.BlockSpec(memory_space=pl.ANY),
        out_shape=jax.ShapeDtypeStruct((N, C), jnp.float32),
        scratch_shapes=[pltpu.VMEM((N, C), jnp.float32)],
    )(y2, mask2, prop)


# speedup vs baseline: 1.1298x; 1.0975x over previous
import functools

import jax
import jax.numpy as jnp
from jax.experimental import pallas as pl
from jax.experimental.pallas import tpu as pltpu

N = 10000
C = 128
BM = 200
NBUF = 4


def _lp_kernel(y_ref, mask_ref, prop_hbm, out_hbm, label_ref):
    classes = jax.lax.broadcasted_iota(jnp.int32, (N, C), 1)
    eq = classes == y_ref[:][:, None]
    maskf = jnp.where(mask_ref[:], 1.0, 0.0)[:, None]
    label_ref[:] = jnp.where(eq, maskf, 0.0)

    def step(prop_blk, out_blk):
        acc = jax.lax.dot_general(
            prop_blk[:],
            label_ref[:],
            (((1,), (0,)), ((), ())),
            preferred_element_type=jnp.float32,
        )
        out_blk[:] = jnp.clip(acc, 0.0, 1.0)

    pipeline = pltpu.emit_pipeline(
        step,
        grid=(N // BM,),
        in_specs=[
            pl.BlockSpec((BM, N), lambda i: (i, 0),
                         pipeline_mode=pl.Buffered(buffer_count=NBUF)),
        ],
        out_specs=[pl.BlockSpec((BM, C), lambda i: (i, 0))],
    )
    pipeline(prop_hbm, out_hbm)


@functools.partial(jax.jit, static_argnames=())
def kernel(x, y, train_mask, prop):
    del x
    return pl.pallas_call(
        _lp_kernel,
        in_specs=[
            pl.BlockSpec((N,), lambda: (0,)),
            pl.BlockSpec((N,), lambda: (0,)),
            pl.BlockSpec(memory_space=pl.ANY),
        ],
        out_specs=pl.BlockSpec(memory_space=pl.ANY),
        out_shape=jax.ShapeDtypeStruct((N, C), jnp.float32),
        scratch_shapes=[pltpu.VMEM((N, C), jnp.float32)],
    )(y, train_mask, prop)


# speedup vs baseline: 1.1525x; 1.0201x over previous
import functools

import jax
import jax.numpy as jnp
from jax.experimental import pallas as pl
from jax.experimental.pallas import tpu as pltpu

N = 10000
C = 128
BM = 200
NBUF = 4


def _lp_kernel(y_ref, mask_ref, prop_hbm, out_hbm, label_ref, flag_ref):
    flag_ref[0] = 0

    def step(prop_blk, out_blk):
        @pl.when(flag_ref[0] == 0)
        def _build_label():
            classes = jax.lax.broadcasted_iota(jnp.int32, (N, C), 1)
            eq = classes == y_ref[:][:, None]
            maskf = jnp.where(mask_ref[:], 1.0, 0.0)[:, None]
            label_ref[:] = jnp.where(eq, maskf, 0.0)
            flag_ref[0] = 1

        acc = jax.lax.dot_general(
            prop_blk[:],
            label_ref[:],
            (((1,), (0,)), ((), ())),
            preferred_element_type=jnp.float32,
        )
        out_blk[:] = jnp.clip(acc, 0.0, 1.0)

    pipeline = pltpu.emit_pipeline(
        step,
        grid=(N // BM,),
        in_specs=[
            pl.BlockSpec((BM, N), lambda i: (i, 0),
                         pipeline_mode=pl.Buffered(buffer_count=NBUF)),
        ],
        out_specs=[pl.BlockSpec((BM, C), lambda i: (i, 0))],
    )
    pipeline(prop_hbm, out_hbm)


@functools.partial(jax.jit, static_argnames=())
def kernel(x, y, train_mask, prop):
    del x
    return pl.pallas_call(
        _lp_kernel,
        in_specs=[
            pl.BlockSpec((N,), lambda: (0,)),
            pl.BlockSpec((N,), lambda: (0,)),
            pl.BlockSpec(memory_space=pl.ANY),
        ],
        out_specs=pl.BlockSpec(memory_space=pl.ANY),
        out_shape=jax.ShapeDtypeStruct((N, C), jnp.float32),
        scratch_shapes=[pltpu.VMEM((N, C), jnp.float32),
                        pltpu.SMEM((1,), jnp.int32)],
    )(y, train_mask, prop)


# speedup vs baseline: 1.1580x; 1.0048x over previous
import functools

import jax
import jax.numpy as jnp
from jax.experimental import pallas as pl
from jax.experimental.pallas import tpu as pltpu

N = 10000
C = 128
BM = 200
NBUF = 4


def _lp_kernel(y_ref, mask_ref, prop_hbm, out_hbm, label_ref, flag_ref):
    flag_ref[0] = 0

    def step(prop_blk, out_blk):
        @pl.when(flag_ref[0] == 0)
        def _build_label():
            classes = jax.lax.broadcasted_iota(jnp.int32, (N, C), 1)
            eq = classes == y_ref[:][:, None]
            maskf = jnp.where(mask_ref[:], 1.0, 0.0)[:, None]
            label_ref[:] = jnp.where(eq, maskf, 0.0).astype(jnp.bfloat16)
            flag_ref[0] = 1

        acc = jax.lax.dot_general(
            prop_blk[:].astype(jnp.bfloat16),
            label_ref[:],
            (((1,), (0,)), ((), ())),
            preferred_element_type=jnp.float32,
        )
        out_blk[:] = jnp.clip(acc, 0.0, 1.0)

    pipeline = pltpu.emit_pipeline(
        step,
        grid=(N // BM,),
        in_specs=[
            pl.BlockSpec((BM, N), lambda i: (i, 0),
                         pipeline_mode=pl.Buffered(buffer_count=NBUF)),
        ],
        out_specs=[pl.BlockSpec((BM, C), lambda i: (i, 0))],
    )
    pipeline(prop_hbm, out_hbm)


@functools.partial(jax.jit, static_argnames=())
def kernel(x, y, train_mask, prop):
    del x
    return pl.pallas_call(
        _lp_kernel,
        in_specs=[
            pl.BlockSpec((N,), lambda: (0,)),
            pl.BlockSpec((N,), lambda: (0,)),
            pl.BlockSpec(memory_space=pl.ANY),
        ],
        out_specs=pl.BlockSpec(memory_space=pl.ANY),
        out_shape=jax.ShapeDtypeStruct((N, C), jnp.float32),
        scratch_shapes=[pltpu.VMEM((N, C), jnp.bfloat16),
                        pltpu.SMEM((1,), jnp.int32)],
    )(y, train_mask, prop)
